# baseline (speedup 1.0000x reference)
import jax
import jax.numpy as jnp
from jax import lax
from jax.experimental import pallas as pl
from jax.experimental.pallas import tpu as pltpu
from jax.experimental.pallas import tpu_sc as plsc

_M, _N = 65536, 1024
_NQ = 64
_PW = 128
_NW = 32
_RPW = _M // _NW
_CH = 128
_NCH = _RPW // _CH
_NPAIR = _NCH // 2


def _fill(buf, chanv, chanbase):
    for t in range(_CH // 16):
        c = chanv[pl.ds(chanbase + t * 16, 16)]
        for j in range(16):
            v = jnp.full((16,), c[j], jnp.float32)
            r = t * 16 + j
            for k in range(_NQ // 16):
                buf[r, pl.ds(k * 16, 16)] = v


def _sc_body(data_hbm, chan_hbm, out_hbm, chanv, bufa, bufb,
             bigsem, ina, inb, outa, outb):
    c = lax.axis_index("c")
    s = lax.axis_index("s")
    wid = s * 2 + c
    base = wid * _RPW
    big = pltpu.async_copy(
        data_hbm.at[pl.ds(base, _RPW), pl.ds(_PW, _N - _PW)],
        out_hbm.at[pl.ds(base, _RPW), pl.ds(_PW, _N - _PW)],
        bigsem,
    )
    pltpu.sync_copy(chan_hbm.at[pl.ds(base, _RPW)], chanv)

    def in_copy(chunk, buf, sem):
        return pltpu.async_copy(
            data_hbm.at[pl.ds(base + chunk * _CH, _CH), pl.ds(0, _PW)],
            buf, sem)

    def in_wait(chunk, buf, sem):
        pltpu.make_async_copy(
            data_hbm.at[pl.ds(base + chunk * _CH, _CH), pl.ds(0, _PW)],
            buf, sem).wait()

    def out_copy(chunk, buf, sem):
        return pltpu.async_copy(
            buf, out_hbm.at[pl.ds(base + chunk * _CH, _CH), pl.ds(0, _PW)],
            sem)

    in_copy(0, bufa, ina)
    in_copy(1, bufb, inb)

    def pair(g, carry):
        ca = 2 * g
        in_wait(ca, bufa, ina)
        _fill(bufa, chanv, ca * _CH)
        oa = out_copy(ca, bufa, outa)
        in_wait(ca + 1, bufb, inb)
        _fill(bufb, chanv, (ca + 1) * _CH)
        ob = out_copy(ca + 1, bufb, outb)

        @pl.when(g < _NPAIR - 1)
        def _prefetch():
            oa.wait()
            in_copy(ca + 2, bufa, ina)
            ob.wait()
            in_copy(ca + 3, bufb, inb)

        @pl.when(g == _NPAIR - 1)
        def _drain():
            oa.wait()
            ob.wait()

        return carry

    lax.fori_loop(0, _NPAIR, pair, 0)
    big.wait()


def kernel(data, query, channel_index):
    del query
    chanf = channel_index.astype(data.dtype)
    mesh = plsc.VectorSubcoreMesh(core_axis_name="c", subcore_axis_name="s")
    f = pl.kernel(
        _sc_body,
        out_type=jax.ShapeDtypeStruct((_M, _N), data.dtype),
        mesh=mesh,
        scratch_types=[
            pltpu.VMEM((_RPW,), jnp.float32),
            pltpu.VMEM((_CH, _PW), jnp.float32),
            pltpu.VMEM((_CH, _PW), jnp.float32),
            pltpu.SemaphoreType.DMA,
            pltpu.SemaphoreType.DMA,
            pltpu.SemaphoreType.DMA,
            pltpu.SemaphoreType.DMA,
            pltpu.SemaphoreType.DMA,
        ],
    )
    return f(data, chanf)


# speedup vs baseline: 33.5354x; 33.5354x over previous
import jax
import jax.numpy as jnp
from jax import lax
from jax.experimental import pallas as pl
from jax.experimental.pallas import tpu as pltpu
from jax.experimental.pallas import tpu_sc as plsc

_M, _N = 65536, 1024
_NQ = 64
_NW = 32
_RPW = _M // _NW
_CH = 32
_NCH = _RPW // _CH
_NPAIR = _NCH // 2


def _fill(buf, chanv, chanbase):
    for t in range(_CH // 16):
        c = chanv[pl.ds(chanbase + t * 16, 16)]
        for j in range(16):
            v = jnp.full((16,), c[j], jnp.float32)
            r = t * 16 + j
            for k in range(_NQ // 16):
                buf[r, pl.ds(k * 16, 16)] = v


def _sc_body(data_hbm, chan_hbm, out_hbm, chanv, bufa, bufb,
             ina, inb, outa, outb):
    c = lax.axis_index("c")
    s = lax.axis_index("s")
    wid = s * 2 + c
    base = wid * _RPW
    pltpu.sync_copy(chan_hbm.at[pl.ds(base, _RPW)], chanv)

    def in_copy(chunk, buf, sem):
        return pltpu.async_copy(
            data_hbm.at[pl.ds(base + chunk * _CH, _CH)], buf, sem)

    def in_wait(chunk, buf, sem):
        pltpu.make_async_copy(
            data_hbm.at[pl.ds(base + chunk * _CH, _CH)], buf, sem).wait()

    def out_copy(chunk, buf, sem):
        return pltpu.async_copy(
            buf, out_hbm.at[pl.ds(base + chunk * _CH, _CH)], sem)

    in_copy(0, bufa, ina)
    in_copy(1, bufb, inb)

    def pair(g, carry):
        ca = 2 * g
        in_wait(ca, bufa, ina)
        _fill(bufa, chanv, ca * _CH)
        oa = out_copy(ca, bufa, outa)
        in_wait(ca + 1, bufb, inb)
        _fill(bufb, chanv, (ca + 1) * _CH)
        ob = out_copy(ca + 1, bufb, outb)

        @pl.when(g < _NPAIR - 1)
        def _prefetch():
            oa.wait()
            in_copy(ca + 2, bufa, ina)
            ob.wait()
            in_copy(ca + 3, bufb, inb)

        @pl.when(g == _NPAIR - 1)
        def _drain():
            oa.wait()
            ob.wait()

        return carry

    lax.fori_loop(0, _NPAIR, pair, 0)


def kernel(data, query, channel_index):
    del query
    chanf = channel_index.astype(data.dtype)
    mesh = plsc.VectorSubcoreMesh(core_axis_name="c", subcore_axis_name="s")
    f = pl.kernel(
        _sc_body,
        out_type=jax.ShapeDtypeStruct((_M, _N), data.dtype),
        mesh=mesh,
        scratch_types=[
            pltpu.VMEM((_RPW,), jnp.float32),
            pltpu.VMEM((_CH, _N), jnp.float32),
            pltpu.VMEM((_CH, _N), jnp.float32),
            pltpu.SemaphoreType.DMA,
            pltpu.SemaphoreType.DMA,
            pltpu.SemaphoreType.DMA,
            pltpu.SemaphoreType.DMA,
        ],
    )
    return f(data, chanf)


# speedup vs baseline: 34.6587x; 1.0335x over previous
import jax
import jax.numpy as jnp
from jax import lax
from jax.experimental import pallas as pl
from jax.experimental.pallas import tpu as pltpu
from jax.experimental.pallas import tpu_sc as plsc

_M, _N = 65536, 1024
_NQ = 64
_NW = 32
_RPW = _M // _NW
_CH = 16
_NBUF = 4
_NCH = _RPW // _CH
_NGRP = _NCH // _NBUF


def _fill(buf, chanv, chanbase):
    for t in range(_CH // 16):
        c = chanv[pl.ds(chanbase + t * 16, 16)]
        for j in range(16):
            v = jnp.full((16,), c[j], jnp.float32)
            r = t * 16 + j
            for k in range(_NQ // 16):
                buf[r, pl.ds(k * 16, 16)] = v


def _sc_body(data_hbm, chan_hbm, out_hbm, chanv,
             buf0, buf1, buf2, buf3,
             in0, in1, in2, in3, ou0, ou1, ou2, ou3):
    bufs = (buf0, buf1, buf2, buf3)
    ins = (in0, in1, in2, in3)
    outs = (ou0, ou1, ou2, ou3)
    c = lax.axis_index("c")
    s = lax.axis_index("s")
    wid = s * 2 + c
    base = wid * _RPW
    pltpu.sync_copy(chan_hbm.at[pl.ds(base, _RPW)], chanv)

    def in_copy(chunk, buf, sem):
        return pltpu.async_copy(
            data_hbm.at[pl.ds(base + chunk * _CH, _CH)], buf, sem)

    def in_wait(chunk, buf, sem):
        pltpu.make_async_copy(
            data_hbm.at[pl.ds(base + chunk * _CH, _CH)], buf, sem).wait()

    def out_copy(chunk, buf, sem):
        return pltpu.async_copy(
            buf, out_hbm.at[pl.ds(base + chunk * _CH, _CH)], sem)

    for b in range(_NBUF):
        in_copy(b, bufs[b], ins[b])

    def group(g, carry):
        ch0 = g * _NBUF
        handles = []
        for b in range(_NBUF):
            ch = ch0 + b
            in_wait(ch, bufs[b], ins[b])
            _fill(bufs[b], chanv, ch * _CH)
            handles.append(out_copy(ch, bufs[b], outs[b]))

        @pl.when(g < _NGRP - 1)
        def _prefetch():
            for b in range(_NBUF):
                handles[b].wait()
                in_copy(ch0 + _NBUF + b, bufs[b], ins[b])

        @pl.when(g == _NGRP - 1)
        def _drain():
            for b in range(_NBUF):
                handles[b].wait()

        return carry

    lax.fori_loop(0, _NGRP, group, 0)


def kernel(data, query, channel_index):
    del query
    chanf = channel_index.astype(data.dtype)
    mesh = plsc.VectorSubcoreMesh(core_axis_name="c", subcore_axis_name="s")
    f = pl.kernel(
        _sc_body,
        out_type=jax.ShapeDtypeStruct((_M, _N), data.dtype),
        mesh=mesh,
        scratch_types=(
            [pltpu.VMEM((_RPW,), jnp.float32)]
            + [pltpu.VMEM((_CH, _N), jnp.float32) for _ in range(_NBUF)]
            + [pltpu.SemaphoreType.DMA for _ in range(2 * _NBUF)]
        ),
    )
    return f(data, chanf)
